# baseline (speedup 1.0000x reference)
import functools

import jax
import jax.numpy as jnp
from jax import lax
from jax.experimental import pallas as pl
from jax.experimental.pallas import tpu as pltpu
from jax.experimental.pallas import tpu_sc as plsc

NC = 2
NS = 16
NW = NC * NS
CHUNK = 40
GRP = 10


def _mesh():
    return plsc.VectorSubcoreMesh(
        core_axis_name="c", subcore_axis_name="s", num_cores=NC, num_subcores=NS
    )


def _pad_rows(n):
    q = NS * 128
    return ((n + q - 1) // q) * q


CE = 16000


def _deg_hist_body(ng, hrows, e, dst_ref, dinv_ref):
    i = pl.program_id(0)

    @pl.when(i == 0)
    def _():
        dinv_ref[...] = jnp.zeros_like(dinv_ref)

    d = dst_ref[0]
    lane = lax.broadcasted_iota(jnp.int32, (128, d.shape[1]), 0)
    grp = lax.broadcasted_iota(jnp.int32, (hrows, d.shape[1]), 0)
    m1 = jnp.where(jnp.bitwise_and(d, 127) == lane,
                   1.0, 0.0).astype(jnp.bfloat16)
    s = jnp.where(lax.shift_right_logical(d, 7) == grp,
                  1.0, 0.0).astype(jnp.bfloat16)
    dinv_ref[...] += lax.dot_general(s, m1, (((1,), (1,)), ((), ())),
                                     preferred_element_type=jnp.float32)

    @pl.when(i == ng - 1)
    def _():
        dinv_ref[...] = lax.rsqrt(dinv_ref[...] + 1.0)


@functools.lru_cache(maxsize=None)
def _deg_hist(n_pad, e):
    ng = e // CE
    hrows = n_pad // 128
    return pl.pallas_call(
        functools.partial(_deg_hist_body, ng, hrows, e),
        grid=(ng,),
        in_specs=[pl.BlockSpec((1, 1, CE), lambda i: (i, 0, 0))],
        out_specs=pl.BlockSpec((hrows, 128), lambda i: (0, 0)),
        out_shape=jax.ShapeDtypeStruct((hrows, 128), jnp.float32),
    )


def _dinv_col(dinv_ref):
    rows = _ROWS // 128
    a = dinv_ref[pl.ds(pl.program_id(0) * rows, rows), :]
    rid = lax.broadcasted_iota(jnp.int32, (_ROWS, rows), 0)
    gid = lax.broadcasted_iota(jnp.int32, (_ROWS, rows), 1)
    oh = jnp.where(lax.shift_right_logical(rid, 7) == gid, 1.0, 0.0)
    b = jnp.dot(oh, a, preferred_element_type=jnp.float32)
    rid2 = lax.broadcasted_iota(jnp.int32, (_ROWS, 128), 0)
    lid = lax.broadcasted_iota(jnp.int32, (_ROWS, 128), 1)
    m = jnp.where(jnp.bitwise_and(rid2, 127) == lid, 1.0, 0.0)
    return jnp.sum(b * m, axis=1, keepdims=True)


def _mm_scale_body(dinv_ref, x_ref, w_ref, y_ref):
    xw = jnp.dot(x_ref[...], w_ref[...], preferred_element_type=jnp.float32)
    y_ref[...] = xw * _dinv_col(dinv_ref)


@functools.lru_cache(maxsize=None)
def _mm_scale(n, d, h, n_pad):
    g = n_pad // _ROWS
    hrows = n_pad // 128
    return pl.pallas_call(
        _mm_scale_body,
        grid=(g,),
        in_specs=[
            pl.BlockSpec((hrows, 128), lambda i: (0, 0)),
            pl.BlockSpec((_ROWS, d), lambda i: (i, 0)),
            pl.BlockSpec((d, h), lambda i: (0, 0)),
        ],
        out_specs=pl.BlockSpec((_ROWS, h), lambda i: (i, 0)),
        out_shape=jax.ShapeDtypeStruct((n_pad, h), jnp.float32),
    )


NBUF = 5
ZCOPY = 40
ZWB = 128


@functools.lru_cache(maxsize=None)
def _edge_scatter(n, h, e):
    cpw = e // (NW * CHUNK)
    epw = cpw * CHUNK
    ngrp = cpw // GRP
    n_pad = _pad_rows(n)
    rps = n_pad // NS
    nz = rps // ZCOPY

    def body(y_hbm, e5_hbm, acc_out, sidx, didx, b0, b1, b2, b3,
             b4, shared_acc, g0, g1, g2, g3, g4, s0, s1, s2, s3, s4):
        c = lax.axis_index("c")
        s = lax.axis_index("s")
        wid = s * NC + c
        bufs = (b0, b1, b2, b3, b4)
        gsem = (g0, g1, g2, g3, g4)
        ssem = (s0, s1, s2, s3, s4)

        pltpu.sync_copy(e5_hbm.at[0, wid, 0], sidx.at[0])
        pltpu.sync_copy(e5_hbm.at[1, wid, 0], didx.at[0])

        @pl.loop(0, CHUNK)
        def _(i):
            for j in range(h // 16):
                b0[i, pl.ds(j * 16, 16)] = jnp.zeros((16,), jnp.float32)

        for k in range(nz):
            pltpu.async_copy(b0.at[pl.ds(0, ZCOPY)],
                             shared_acc.at[pl.ds(s * rps + k * ZCOPY, ZCOPY)],
                             s0)
        for k in range(nz):
            pltpu.make_async_copy(
                b0.at[pl.ds(0, ZCOPY)],
                shared_acc.at[pl.ds(s * rps + k * ZCOPY, ZCOPY)], s0).wait()
        plsc.subcore_barrier()

        def gather(jj, p):
            gg = jj // GRP
            pltpu.async_copy(
                y_hbm.at[sidx.at[lax.rem(gg, 2), jj - gg * GRP]],
                bufs[p], gsem[p])

        for p in range(NBUF):
            gather(p, p)

        @pl.loop(0, cpw // NBUF)
        def _(i):
            jb = i * NBUF
            for p in range(NBUF):
                j = jb + p
                grp = j // GRP
                r = j - grp * GRP
                par = lax.rem(grp, 2)
                if p == 0:
                    @pl.when(r == 0)
                    def _():
                        pltpu.sync_copy(e5_hbm.at[1, wid, grp], didx.at[par])

                        @pl.when(grp + 1 < ngrp)
                        def _():
                            pltpu.sync_copy(e5_hbm.at[0, wid, grp + 1],
                                            sidx.at[1 - par])
                q = (p + NBUF - 1) % NBUF
                gg = j // GRP
                pltpu.make_async_copy(
                    y_hbm.at[sidx.at[lax.rem(gg, 2), j - gg * GRP]],
                    bufs[p], gsem[p]).wait()
                pltpu.async_copy(bufs[p], shared_acc.at[didx.at[par, r]],
                                 ssem[p], add=True)

                @pl.when((j >= 1) & (j + NBUF - 1 < cpw))
                def _():
                    jm = j - 1
                    gm = jm // GRP
                    pltpu.make_async_copy(
                        bufs[q],
                        shared_acc.at[didx.at[lax.rem(gm, 2), jm - gm * GRP]],
                        ssem[q]).wait()
                    gather(j + NBUF - 1, q)

        for p in range(NBUF):
            j = cpw - NBUF + p
            pltpu.make_async_copy(
                bufs[p],
                shared_acc.at[didx.at[(j // GRP) % 2, j % GRP]],
                ssem[p]).wait()

        plsc.subcore_barrier()
        for k in range(rps // ZWB):
            sl = pl.ds(s * rps + k * ZWB, ZWB)
            pltpu.async_copy(shared_acc.at[sl], acc_out.at[c, sl], s0)
        for k in range(rps // ZWB):
            sl = pl.ds(s * rps + k * ZWB, ZWB)
            pltpu.make_async_copy(shared_acc.at[sl], acc_out.at[c, sl],
                                  s0).wait()

    return pl.kernel(
        body,
        out_type=jax.ShapeDtypeStruct((NC, n_pad, h), jnp.float32),
        mesh=_mesh(),
        scratch_types=[
            pltpu.VMEM((2, GRP, CHUNK), jnp.int32),
            pltpu.VMEM((2, GRP, CHUNK), jnp.int32),
            pltpu.VMEM((CHUNK, h), jnp.float32),
            pltpu.VMEM((CHUNK, h), jnp.float32),
            pltpu.VMEM((CHUNK, h), jnp.float32),
            pltpu.VMEM((CHUNK, h), jnp.float32),
            pltpu.VMEM((CHUNK, h), jnp.float32),
            pltpu.VMEM_SHARED((n_pad, h), jnp.float32),
            pltpu.SemaphoreType.DMA,
            pltpu.SemaphoreType.DMA,
            pltpu.SemaphoreType.DMA,
            pltpu.SemaphoreType.DMA,
            pltpu.SemaphoreType.DMA,
            pltpu.SemaphoreType.DMA,
            pltpu.SemaphoreType.DMA,
            pltpu.SemaphoreType.DMA,
            pltpu.SemaphoreType.DMA,
            pltpu.SemaphoreType.DMA,
        ],
    )


_ROWS = 2048


def _layer_mid_body(acc_ref, y_ref, dinv_ref, b_ref, w_ref, h_ref, y2_ref):
    dinv = _dinv_col(dinv_ref)
    agg = acc_ref[0] + acc_ref[1] + y_ref[...]
    hpre = agg * dinv + b_ref[...]
    hh = jnp.maximum(hpre, 0.0)
    h_ref[...] = hh
    y2_ref[...] = jnp.dot(hh, w_ref[...], preferred_element_type=jnp.float32) * dinv


@functools.lru_cache(maxsize=None)
def _layer_mid(n, h, n_pad):
    g = n_pad // _ROWS
    hrows = n_pad // 128
    return pl.pallas_call(
        _layer_mid_body,
        grid=(g,),
        in_specs=[
            pl.BlockSpec((NC, _ROWS, h), lambda i: (0, i, 0)),
            pl.BlockSpec((_ROWS, h), lambda i: (i, 0)),
            pl.BlockSpec((hrows, 128), lambda i: (0, 0)),
            pl.BlockSpec((1, h), lambda i: (0, 0)),
            pl.BlockSpec((h, h), lambda i: (0, 0)),
        ],
        out_specs=[
            pl.BlockSpec((_ROWS, h), lambda i: (i, 0)),
            pl.BlockSpec((_ROWS, h), lambda i: (i, 0)),
        ],
        out_shape=[
            jax.ShapeDtypeStruct((n_pad, h), jnp.float32),
            jax.ShapeDtypeStruct((n_pad, h), jnp.float32),
        ],
    )


def _layer_out_body(acc_ref, y2_ref, dinv_ref, b_ref, hres_ref, wl_ref, bl_ref,
                    out_ref):
    dinv = _dinv_col(dinv_ref)
    agg = acc_ref[0] + acc_ref[1] + y2_ref[...]
    h2 = jnp.maximum(agg * dinv + b_ref[...], 0.0) + hres_ref[...]
    logits = jnp.dot(h2, wl_ref[...], preferred_element_type=jnp.float32)
    logits = logits + bl_ref[...]
    m = jnp.max(logits, axis=1, keepdims=True)
    z = logits - m
    lse = jnp.log(jnp.sum(jnp.exp(z), axis=1, keepdims=True))
    out_ref[...] = z - lse


@functools.lru_cache(maxsize=None)
def _layer_out(n, h, cdim, n_pad):
    g = n_pad // _ROWS
    hrows = n_pad // 128
    return pl.pallas_call(
        _layer_out_body,
        grid=(g,),
        in_specs=[
            pl.BlockSpec((NC, _ROWS, h), lambda i: (0, i, 0)),
            pl.BlockSpec((_ROWS, h), lambda i: (i, 0)),
            pl.BlockSpec((hrows, 128), lambda i: (0, 0)),
            pl.BlockSpec((1, h), lambda i: (0, 0)),
            pl.BlockSpec((_ROWS, h), lambda i: (i, 0)),
            pl.BlockSpec((h, cdim), lambda i: (0, 0)),
            pl.BlockSpec((1, cdim), lambda i: (0, 0)),
        ],
        out_specs=pl.BlockSpec((_ROWS, cdim), lambda i: (i, 0)),
        out_shape=jax.ShapeDtypeStruct((n, cdim), jnp.float32),
    )


def kernel(x, edge_index, W1, b1, W2, b2, Wl, bl):
    n, d = x.shape
    h = W1.shape[1]
    cdim = Wl.shape[1]
    e = edge_index.shape[1]
    cpw = e // (NW * CHUNK)
    n_pad = _pad_rows(n)

    e5 = edge_index.reshape(2, NW, cpw // GRP, GRP, CHUNK)
    dsth = edge_index[1].reshape(e // CE, 1, CE)

    dinv2d = _deg_hist(n_pad, e)(dsth)
    y1 = _mm_scale(n, d, h, n_pad)(dinv2d, x, W1)
    acc1 = _edge_scatter(n, h, e)(y1, e5)
    h1, y2 = _layer_mid(n, h, n_pad)(acc1, y1, dinv2d, b1.reshape(1, h), W2)
    acc2 = _edge_scatter(n, h, e)(y2, e5)
    out = _layer_out(n, h, cdim, n_pad)(acc2, y2, dinv2d, b2.reshape(1, h),
                                        h1, Wl, bl.reshape(1, cdim))
    return out


# speedup vs baseline: 1.0676x; 1.0676x over previous
import functools

import jax
import jax.numpy as jnp
from jax import lax
from jax.experimental import pallas as pl
from jax.experimental.pallas import tpu as pltpu
from jax.experimental.pallas import tpu_sc as plsc

NC = 2
NS = 16
NW = NC * NS
CHUNK = 40
GRP = 10


def _mesh():
    return plsc.VectorSubcoreMesh(
        core_axis_name="c", subcore_axis_name="s", num_cores=NC, num_subcores=NS
    )


def _pad_rows(n):
    q = NS * 128
    return ((n + q - 1) // q) * q


CE = 16000


def _deg_hist_body(ng, hrows, e, dst_ref, dinv_ref):
    i = pl.program_id(0)

    @pl.when(i == 0)
    def _():
        dinv_ref[...] = jnp.zeros_like(dinv_ref)

    d = dst_ref[0]
    lane = lax.broadcasted_iota(jnp.int32, (128, d.shape[1]), 0)
    grp = lax.broadcasted_iota(jnp.int32, (hrows, d.shape[1]), 0)
    m1 = jnp.where(jnp.bitwise_and(d, 127) == lane,
                   1.0, 0.0).astype(jnp.bfloat16)
    s = jnp.where(lax.shift_right_logical(d, 7) == grp,
                  1.0, 0.0).astype(jnp.bfloat16)
    dinv_ref[...] += lax.dot_general(s, m1, (((1,), (1,)), ((), ())),
                                     preferred_element_type=jnp.float32)

    @pl.when(i == ng - 1)
    def _():
        dinv_ref[...] = lax.rsqrt(dinv_ref[...] + 1.0)


@functools.lru_cache(maxsize=None)
def _deg_hist(n_pad, e):
    ng = e // CE
    hrows = n_pad // 128
    return pl.pallas_call(
        functools.partial(_deg_hist_body, ng, hrows, e),
        grid=(ng,),
        in_specs=[pl.BlockSpec((1, 1, CE), lambda i: (i, 0, 0))],
        out_specs=pl.BlockSpec((hrows, 128), lambda i: (0, 0)),
        out_shape=jax.ShapeDtypeStruct((hrows, 128), jnp.float32),
    )


def _dinv_col(dinv_ref):
    rows = _ROWS // 128
    a = dinv_ref[pl.ds(pl.program_id(0) * rows, rows), :]
    rid = lax.broadcasted_iota(jnp.int32, (_ROWS, rows), 0)
    gid = lax.broadcasted_iota(jnp.int32, (_ROWS, rows), 1)
    oh = jnp.where(lax.shift_right_logical(rid, 7) == gid, 1.0, 0.0)
    b = jnp.dot(oh, a, preferred_element_type=jnp.float32)
    rid2 = lax.broadcasted_iota(jnp.int32, (_ROWS, 128), 0)
    lid = lax.broadcasted_iota(jnp.int32, (_ROWS, 128), 1)
    m = jnp.where(jnp.bitwise_and(rid2, 127) == lid, 1.0, 0.0)
    return jnp.sum(b * m, axis=1, keepdims=True)


def _mm_scale_body(dinv_ref, x_ref, w_ref, y_ref):
    xw = jnp.dot(x_ref[...], w_ref[...], preferred_element_type=jnp.float32)
    y_ref[...] = xw * _dinv_col(dinv_ref)


@functools.lru_cache(maxsize=None)
def _mm_scale(n, d, h, n_pad):
    g = n_pad // _ROWS
    hrows = n_pad // 128
    return pl.pallas_call(
        _mm_scale_body,
        grid=(g,),
        in_specs=[
            pl.BlockSpec((hrows, 128), lambda i: (0, 0)),
            pl.BlockSpec((_ROWS, d), lambda i: (i, 0)),
            pl.BlockSpec((d, h), lambda i: (0, 0)),
        ],
        out_specs=pl.BlockSpec((_ROWS, h), lambda i: (i, 0)),
        out_shape=jax.ShapeDtypeStruct((n_pad, h), jnp.float32),
    )


NBUF = 5
ZCOPY = 40
ZWB = 128


@functools.lru_cache(maxsize=None)
def _edge_scatter(n, h, e):
    cpw = e // (NW * CHUNK)
    epw = cpw * CHUNK
    ngrp = cpw // GRP
    n_pad = _pad_rows(n)
    rps = n_pad // NS
    nz = rps // ZCOPY

    def body(y_hbm, src1_hbm, dst4_hbm, acc_out, sidx, didx, b0, b1, b2, b3,
             b4, shared_acc, g0, g1, g2, g3, g4, s0, s1, s2, s3, s4):
        c = lax.axis_index("c")
        s = lax.axis_index("s")
        wid = s * NC + c
        bufs = (b0, b1, b2, b3, b4)
        gsem = (g0, g1, g2, g3, g4)
        ssem = (s0, s1, s2, s3, s4)

        idx_load = pltpu.async_copy(src1_hbm.at[pl.ds(wid * epw, epw)], sidx,
                                    g0)
        pltpu.sync_copy(dst4_hbm.at[wid, 0], didx.at[0])

        @pl.loop(0, CHUNK)
        def _(i):
            for j in range(h // 16):
                b0[i, pl.ds(j * 16, 16)] = jnp.zeros((16,), jnp.float32)

        for k in range(nz):
            pltpu.async_copy(b0.at[pl.ds(0, ZCOPY)],
                             shared_acc.at[pl.ds(s * rps + k * ZCOPY, ZCOPY)],
                             s0)
        for k in range(nz):
            pltpu.make_async_copy(
                b0.at[pl.ds(0, ZCOPY)],
                shared_acc.at[pl.ds(s * rps + k * ZCOPY, ZCOPY)], s0).wait()
        idx_load.wait()
        plsc.subcore_barrier()

        def gather(jj, p):
            pltpu.async_copy(y_hbm.at[sidx.at[pl.ds(jj * CHUNK, CHUNK)]],
                             bufs[p], gsem[p])

        for p in range(NBUF):
            gather(p, p)

        @pl.loop(0, cpw // NBUF)
        def _(i):
            jb = i * NBUF
            for p in range(NBUF):
                j = jb + p
                grp = j // GRP
                r = j - grp * GRP
                par = lax.rem(grp, 2)
                if p == 0:
                    @pl.when((r == 0) & (grp > 0))
                    def _():
                        pltpu.sync_copy(dst4_hbm.at[wid, grp], didx.at[par])
                q = (p + NBUF - 1) % NBUF
                pltpu.make_async_copy(
                    y_hbm.at[sidx.at[pl.ds(j * CHUNK, CHUNK)]],
                    bufs[p], gsem[p]).wait()
                pltpu.async_copy(bufs[p], shared_acc.at[didx.at[par, r]],
                                 ssem[p], add=True)

                @pl.when((j >= 1) & (j + NBUF - 1 < cpw))
                def _():
                    jm = j - 1
                    gm = jm // GRP
                    pltpu.make_async_copy(
                        bufs[q],
                        shared_acc.at[didx.at[lax.rem(gm, 2), jm - gm * GRP]],
                        ssem[q]).wait()
                    gather(j + NBUF - 1, q)

        for p in range(NBUF):
            j = cpw - NBUF + p
            pltpu.make_async_copy(
                bufs[p],
                shared_acc.at[didx.at[(j // GRP) % 2, j % GRP]],
                ssem[p]).wait()

        plsc.subcore_barrier()
        for k in range(rps // ZWB):
            sl = pl.ds(s * rps + k * ZWB, ZWB)
            pltpu.async_copy(shared_acc.at[sl], acc_out.at[c, sl], s0)
        for k in range(rps // ZWB):
            sl = pl.ds(s * rps + k * ZWB, ZWB)
            pltpu.make_async_copy(shared_acc.at[sl], acc_out.at[c, sl],
                                  s0).wait()

    return pl.kernel(
        body,
        out_type=jax.ShapeDtypeStruct((NC, n_pad, h), jnp.float32),
        mesh=_mesh(),
        scratch_types=[
            pltpu.VMEM((epw,), jnp.int32),
            pltpu.VMEM((2, GRP, CHUNK), jnp.int32),
            pltpu.VMEM((CHUNK, h), jnp.float32),
            pltpu.VMEM((CHUNK, h), jnp.float32),
            pltpu.VMEM((CHUNK, h), jnp.float32),
            pltpu.VMEM((CHUNK, h), jnp.float32),
            pltpu.VMEM((CHUNK, h), jnp.float32),
            pltpu.VMEM_SHARED((n_pad, h), jnp.float32),
            pltpu.SemaphoreType.DMA,
            pltpu.SemaphoreType.DMA,
            pltpu.SemaphoreType.DMA,
            pltpu.SemaphoreType.DMA,
            pltpu.SemaphoreType.DMA,
            pltpu.SemaphoreType.DMA,
            pltpu.SemaphoreType.DMA,
            pltpu.SemaphoreType.DMA,
            pltpu.SemaphoreType.DMA,
            pltpu.SemaphoreType.DMA,
        ],
    )


_ROWS = 2048


def _layer_mid_body(acc_ref, y_ref, dinv_ref, b_ref, w_ref, h_ref, y2_ref):
    dinv = _dinv_col(dinv_ref)
    agg = acc_ref[0] + acc_ref[1] + y_ref[...]
    hpre = agg * dinv + b_ref[...]
    hh = jnp.maximum(hpre, 0.0)
    h_ref[...] = hh
    y2_ref[...] = jnp.dot(hh, w_ref[...], preferred_element_type=jnp.float32) * dinv


@functools.lru_cache(maxsize=None)
def _layer_mid(n, h, n_pad):
    g = n_pad // _ROWS
    hrows = n_pad // 128
    return pl.pallas_call(
        _layer_mid_body,
        grid=(g,),
        in_specs=[
            pl.BlockSpec((NC, _ROWS, h), lambda i: (0, i, 0)),
            pl.BlockSpec((_ROWS, h), lambda i: (i, 0)),
            pl.BlockSpec((hrows, 128), lambda i: (0, 0)),
            pl.BlockSpec((1, h), lambda i: (0, 0)),
            pl.BlockSpec((h, h), lambda i: (0, 0)),
        ],
        out_specs=[
            pl.BlockSpec((_ROWS, h), lambda i: (i, 0)),
            pl.BlockSpec((_ROWS, h), lambda i: (i, 0)),
        ],
        out_shape=[
            jax.ShapeDtypeStruct((n_pad, h), jnp.float32),
            jax.ShapeDtypeStruct((n_pad, h), jnp.float32),
        ],
    )


def _layer_out_body(acc_ref, y2_ref, dinv_ref, b_ref, hres_ref, wl_ref, bl_ref,
                    out_ref):
    dinv = _dinv_col(dinv_ref)
    agg = acc_ref[0] + acc_ref[1] + y2_ref[...]
    h2 = jnp.maximum(agg * dinv + b_ref[...], 0.0) + hres_ref[...]
    logits = jnp.dot(h2, wl_ref[...], preferred_element_type=jnp.float32)
    logits = logits + bl_ref[...]
    m = jnp.max(logits, axis=1, keepdims=True)
    z = logits - m
    lse = jnp.log(jnp.sum(jnp.exp(z), axis=1, keepdims=True))
    out_ref[...] = z - lse


@functools.lru_cache(maxsize=None)
def _layer_out(n, h, cdim, n_pad):
    g = n_pad // _ROWS
    hrows = n_pad // 128
    return pl.pallas_call(
        _layer_out_body,
        grid=(g,),
        in_specs=[
            pl.BlockSpec((NC, _ROWS, h), lambda i: (0, i, 0)),
            pl.BlockSpec((_ROWS, h), lambda i: (i, 0)),
            pl.BlockSpec((hrows, 128), lambda i: (0, 0)),
            pl.BlockSpec((1, h), lambda i: (0, 0)),
            pl.BlockSpec((_ROWS, h), lambda i: (i, 0)),
            pl.BlockSpec((h, cdim), lambda i: (0, 0)),
            pl.BlockSpec((1, cdim), lambda i: (0, 0)),
        ],
        out_specs=pl.BlockSpec((_ROWS, cdim), lambda i: (i, 0)),
        out_shape=jax.ShapeDtypeStruct((n, cdim), jnp.float32),
    )


def kernel(x, edge_index, W1, b1, W2, b2, Wl, bl):
    n, d = x.shape
    h = W1.shape[1]
    cdim = Wl.shape[1]
    e = edge_index.shape[1]
    cpw = e // (NW * CHUNK)
    n_pad = _pad_rows(n)

    src1 = edge_index[0]
    dst1 = edge_index[1]
    dst4 = dst1.reshape(NW, cpw // GRP, GRP, CHUNK)
    dsth = dst1.reshape(e // CE, 1, CE)

    dinv2d = _deg_hist(n_pad, e)(dsth)
    y1 = _mm_scale(n, d, h, n_pad)(dinv2d, x, W1)
    acc1 = _edge_scatter(n, h, e)(y1, src1, dst4)
    h1, y2 = _layer_mid(n, h, n_pad)(acc1, y1, dinv2d, b1.reshape(1, h), W2)
    acc2 = _edge_scatter(n, h, e)(y2, src1, dst4)
    out = _layer_out(n, h, cdim, n_pad)(acc2, y2, dinv2d, b2.reshape(1, h),
                                        h1, Wl, bl.reshape(1, cdim))
    return out


# speedup vs baseline: 1.1194x; 1.0486x over previous
import functools

import jax
import jax.numpy as jnp
from jax import lax
from jax.experimental import pallas as pl
from jax.experimental.pallas import tpu as pltpu
from jax.experimental.pallas import tpu_sc as plsc

NC = 2
NS = 16
NW = NC * NS
CHUNK = 40
GRP = 25


def _mesh():
    return plsc.VectorSubcoreMesh(
        core_axis_name="c", subcore_axis_name="s", num_cores=NC, num_subcores=NS
    )


def _pad_rows(n):
    q = NS * 128
    return ((n + q - 1) // q) * q


CE = 16000


def _deg_hist_body(ng, hrows, e, dst_ref, dinv_ref):
    i = pl.program_id(0)

    @pl.when(i == 0)
    def _():
        dinv_ref[...] = jnp.zeros_like(dinv_ref)

    d = dst_ref[0]
    lane = lax.broadcasted_iota(jnp.int32, (128, d.shape[1]), 0)
    grp = lax.broadcasted_iota(jnp.int32, (hrows, d.shape[1]), 0)
    m1 = jnp.where(jnp.bitwise_and(d, 127) == lane,
                   1.0, 0.0).astype(jnp.bfloat16)
    s = jnp.where(lax.shift_right_logical(d, 7) == grp,
                  1.0, 0.0).astype(jnp.bfloat16)
    dinv_ref[...] += lax.dot_general(s, m1, (((1,), (1,)), ((), ())),
                                     preferred_element_type=jnp.float32)

    @pl.when(i == ng - 1)
    def _():
        dinv_ref[...] = lax.rsqrt(dinv_ref[...] + 1.0)


@functools.lru_cache(maxsize=None)
def _deg_hist(n_pad, e):
    ng = e // CE
    hrows = n_pad // 128
    return pl.pallas_call(
        functools.partial(_deg_hist_body, ng, hrows, e),
        grid=(ng,),
        in_specs=[pl.BlockSpec((1, 1, CE), lambda i: (i, 0, 0))],
        out_specs=pl.BlockSpec((hrows, 128), lambda i: (0, 0)),
        out_shape=jax.ShapeDtypeStruct((hrows, 128), jnp.float32),
    )


def _dinv_col(dinv_ref):
    rows = _ROWS // 128
    a = dinv_ref[pl.ds(pl.program_id(0) * rows, rows), :]
    rid = lax.broadcasted_iota(jnp.int32, (_ROWS, rows), 0)
    gid = lax.broadcasted_iota(jnp.int32, (_ROWS, rows), 1)
    oh = jnp.where(lax.shift_right_logical(rid, 7) == gid, 1.0, 0.0)
    b = jnp.dot(oh, a, preferred_element_type=jnp.float32)
    rid2 = lax.broadcasted_iota(jnp.int32, (_ROWS, 128), 0)
    lid = lax.broadcasted_iota(jnp.int32, (_ROWS, 128), 1)
    m = jnp.where(jnp.bitwise_and(rid2, 127) == lid, 1.0, 0.0)
    return jnp.sum(b * m, axis=1, keepdims=True)


def _mm_scale_body(dinv_ref, x_ref, w_ref, y_ref):
    xw = jnp.dot(x_ref[...], w_ref[...], preferred_element_type=jnp.float32)
    y_ref[...] = xw * _dinv_col(dinv_ref)


@functools.lru_cache(maxsize=None)
def _mm_scale(n, d, h, n_pad):
    g = n_pad // _ROWS
    hrows = n_pad // 128
    return pl.pallas_call(
        _mm_scale_body,
        grid=(g,),
        in_specs=[
            pl.BlockSpec((hrows, 128), lambda i: (0, 0)),
            pl.BlockSpec((_ROWS, d), lambda i: (i, 0)),
            pl.BlockSpec((d, h), lambda i: (0, 0)),
        ],
        out_specs=pl.BlockSpec((_ROWS, h), lambda i: (i, 0)),
        out_shape=jax.ShapeDtypeStruct((n_pad, h), jnp.float32),
    )


NBUF = 5
ZCOPY = 40
ZWB = 128


@functools.lru_cache(maxsize=None)
def _edge_scatter(n, h, e):
    cpw = e // (NW * CHUNK)
    epw = cpw * CHUNK
    ngrp = cpw // GRP
    n_pad = _pad_rows(n)
    rps = n_pad // NS
    nz = rps // ZCOPY

    def body(y_hbm, src1_hbm, dst4_hbm, acc_out, sidx, didx, b0, b1, b2, b3,
             b4, shared_acc, g0, g1, g2, g3, g4, s0, s1, s2, s3, s4, dsem):
        c = lax.axis_index("c")
        s = lax.axis_index("s")
        wid = s * NC + c
        bufs = (b0, b1, b2, b3, b4)
        gsem = (g0, g1, g2, g3, g4)
        ssem = (s0, s1, s2, s3, s4)

        idx_load = pltpu.async_copy(src1_hbm.at[pl.ds(wid * epw, epw)], sidx,
                                    g0)
        pltpu.sync_copy(dst4_hbm.at[wid, 0], didx.at[0])
        if ngrp > 1:
            pltpu.async_copy(dst4_hbm.at[wid, 1], didx.at[1], dsem)

        @pl.loop(0, CHUNK)
        def _(i):
            for j in range(h // 16):
                b0[i, pl.ds(j * 16, 16)] = jnp.zeros((16,), jnp.float32)

        for k in range(nz):
            pltpu.async_copy(b0.at[pl.ds(0, ZCOPY)],
                             shared_acc.at[pl.ds(s * rps + k * ZCOPY, ZCOPY)],
                             s0)
        for k in range(nz):
            pltpu.make_async_copy(
                b0.at[pl.ds(0, ZCOPY)],
                shared_acc.at[pl.ds(s * rps + k * ZCOPY, ZCOPY)], s0).wait()
        idx_load.wait()

        def gather(jj, p):
            pltpu.async_copy(y_hbm.at[sidx.at[pl.ds(jj * CHUNK, CHUNK)]],
                             bufs[p], gsem[p])

        for p in range(NBUF):
            gather(p, p)
        plsc.subcore_barrier()

        @pl.loop(0, cpw // NBUF)
        def _(i):
            jb = i * NBUF
            for p in range(NBUF):
                j = jb + p
                grp = j // GRP
                r = j - grp * GRP
                par = lax.rem(grp, 2)
                if p == 0:
                    @pl.when((r == 0) & (grp >= 1))
                    def _():
                        pltpu.make_async_copy(dst4_hbm.at[wid, grp],
                                              didx.at[par], dsem).wait()
                if p == 4:
                    @pl.when((r == 4) & (grp + 1 < ngrp))
                    def _():
                        pltpu.async_copy(dst4_hbm.at[wid, grp + 1],
                                        didx.at[1 - par], dsem)
                q = (p + NBUF - 1) % NBUF
                pltpu.make_async_copy(
                    y_hbm.at[sidx.at[pl.ds(j * CHUNK, CHUNK)]],
                    bufs[p], gsem[p]).wait()
                pltpu.async_copy(bufs[p], shared_acc.at[didx.at[par, r]],
                                 ssem[p], add=True)

                @pl.when((j >= 1) & (j + NBUF - 1 < cpw))
                def _():
                    jm = j - 1
                    gm = jm // GRP
                    pltpu.make_async_copy(
                        bufs[q],
                        shared_acc.at[didx.at[lax.rem(gm, 2), jm - gm * GRP]],
                        ssem[q]).wait()
                    gather(j + NBUF - 1, q)

        for p in range(NBUF):
            j = cpw - NBUF + p
            pltpu.make_async_copy(
                bufs[p],
                shared_acc.at[didx.at[(j // GRP) % 2, j % GRP]],
                ssem[p]).wait()

        plsc.subcore_barrier()
        for k in range(rps // ZWB):
            sl = pl.ds(s * rps + k * ZWB, ZWB)
            pltpu.async_copy(shared_acc.at[sl], acc_out.at[c, sl], s0)
        for k in range(rps // ZWB):
            sl = pl.ds(s * rps + k * ZWB, ZWB)
            pltpu.make_async_copy(shared_acc.at[sl], acc_out.at[c, sl],
                                  s0).wait()

    return pl.kernel(
        body,
        out_type=jax.ShapeDtypeStruct((NC, n_pad, h), jnp.float32),
        mesh=_mesh(),
        scratch_types=[
            pltpu.VMEM((epw,), jnp.int32),
            pltpu.VMEM((2, GRP, CHUNK), jnp.int32),
            pltpu.VMEM((CHUNK, h), jnp.float32),
            pltpu.VMEM((CHUNK, h), jnp.float32),
            pltpu.VMEM((CHUNK, h), jnp.float32),
            pltpu.VMEM((CHUNK, h), jnp.float32),
            pltpu.VMEM((CHUNK, h), jnp.float32),
            pltpu.VMEM_SHARED((n_pad, h), jnp.float32),
            pltpu.SemaphoreType.DMA,
            pltpu.SemaphoreType.DMA,
            pltpu.SemaphoreType.DMA,
            pltpu.SemaphoreType.DMA,
            pltpu.SemaphoreType.DMA,
            pltpu.SemaphoreType.DMA,
            pltpu.SemaphoreType.DMA,
            pltpu.SemaphoreType.DMA,
            pltpu.SemaphoreType.DMA,
            pltpu.SemaphoreType.DMA,
            pltpu.SemaphoreType.DMA,
        ],
    )


_ROWS = 2048


def _layer_mid_body(acc_ref, y_ref, dinv_ref, b_ref, w_ref, h_ref, y2_ref):
    dinv = _dinv_col(dinv_ref)
    agg = acc_ref[0] + acc_ref[1] + y_ref[...]
    hpre = agg * dinv + b_ref[...]
    hh = jnp.maximum(hpre, 0.0)
    h_ref[...] = hh
    y2_ref[...] = jnp.dot(hh, w_ref[...], preferred_element_type=jnp.float32) * dinv


@functools.lru_cache(maxsize=None)
def _layer_mid(n, h, n_pad):
    g = n_pad // _ROWS
    hrows = n_pad // 128
    return pl.pallas_call(
        _layer_mid_body,
        grid=(g,),
        in_specs=[
            pl.BlockSpec((NC, _ROWS, h), lambda i: (0, i, 0)),
            pl.BlockSpec((_ROWS, h), lambda i: (i, 0)),
            pl.BlockSpec((hrows, 128), lambda i: (0, 0)),
            pl.BlockSpec((1, h), lambda i: (0, 0)),
            pl.BlockSpec((h, h), lambda i: (0, 0)),
        ],
        out_specs=[
            pl.BlockSpec((_ROWS, h), lambda i: (i, 0)),
            pl.BlockSpec((_ROWS, h), lambda i: (i, 0)),
        ],
        out_shape=[
            jax.ShapeDtypeStruct((n_pad, h), jnp.float32),
            jax.ShapeDtypeStruct((n_pad, h), jnp.float32),
        ],
    )


def _layer_out_body(acc_ref, y2_ref, dinv_ref, b_ref, hres_ref, wl_ref, bl_ref,
                    out_ref):
    dinv = _dinv_col(dinv_ref)
    agg = acc_ref[0] + acc_ref[1] + y2_ref[...]
    h2 = jnp.maximum(agg * dinv + b_ref[...], 0.0) + hres_ref[...]
    logits = jnp.dot(h2, wl_ref[...], preferred_element_type=jnp.float32)
    logits = logits + bl_ref[...]
    m = jnp.max(logits, axis=1, keepdims=True)
    z = logits - m
    lse = jnp.log(jnp.sum(jnp.exp(z), axis=1, keepdims=True))
    out_ref[...] = z - lse


@functools.lru_cache(maxsize=None)
def _layer_out(n, h, cdim, n_pad):
    g = n_pad // _ROWS
    hrows = n_pad // 128
    return pl.pallas_call(
        _layer_out_body,
        grid=(g,),
        in_specs=[
            pl.BlockSpec((NC, _ROWS, h), lambda i: (0, i, 0)),
            pl.BlockSpec((_ROWS, h), lambda i: (i, 0)),
            pl.BlockSpec((hrows, 128), lambda i: (0, 0)),
            pl.BlockSpec((1, h), lambda i: (0, 0)),
            pl.BlockSpec((_ROWS, h), lambda i: (i, 0)),
            pl.BlockSpec((h, cdim), lambda i: (0, 0)),
            pl.BlockSpec((1, cdim), lambda i: (0, 0)),
        ],
        out_specs=pl.BlockSpec((_ROWS, cdim), lambda i: (i, 0)),
        out_shape=jax.ShapeDtypeStruct((n, cdim), jnp.float32),
    )


def kernel(x, edge_index, W1, b1, W2, b2, Wl, bl):
    n, d = x.shape
    h = W1.shape[1]
    cdim = Wl.shape[1]
    e = edge_index.shape[1]
    cpw = e // (NW * CHUNK)
    n_pad = _pad_rows(n)

    src1 = edge_index[0]
    dst1 = edge_index[1]
    dst4 = dst1.reshape(NW, cpw // GRP, GRP, CHUNK)
    dsth = dst1.reshape(e // CE, 1, CE)

    dinv2d = _deg_hist(n_pad, e)(dsth)
    y1 = _mm_scale(n, d, h, n_pad)(dinv2d, x, W1)
    acc1 = _edge_scatter(n, h, e)(y1, src1, dst4)
    h1, y2 = _layer_mid(n, h, n_pad)(acc1, y1, dinv2d, b1.reshape(1, h), W2)
    acc2 = _edge_scatter(n, h, e)(y2, src1, dst4)
    out = _layer_out(n, h, cdim, n_pad)(acc2, y2, dinv2d, b2.reshape(1, h),
                                        h1, Wl, bl.reshape(1, cdim))
    return out


# speedup vs baseline: 1.1267x; 1.0065x over previous
import functools

import jax
import jax.numpy as jnp
from jax import lax
from jax.experimental import pallas as pl
from jax.experimental.pallas import tpu as pltpu
from jax.experimental.pallas import tpu_sc as plsc

NC = 2
NS = 16
NW = NC * NS
CHUNK = 40
GRP = 25


def _mesh():
    return plsc.VectorSubcoreMesh(
        core_axis_name="c", subcore_axis_name="s", num_cores=NC, num_subcores=NS
    )


def _pad_rows(n):
    q = NS * 128
    return ((n + q - 1) // q) * q


CE = 32000


def _deg_hist_body(ng, hrows, e, dst_ref, dinv_ref):
    i = pl.program_id(0)

    @pl.when(i == 0)
    def _():
        dinv_ref[...] = jnp.zeros_like(dinv_ref)

    d = dst_ref[0]
    lane = lax.broadcasted_iota(jnp.int32, (128, d.shape[1]), 0)
    grp = lax.broadcasted_iota(jnp.int32, (hrows, d.shape[1]), 0)
    m1 = jnp.where(jnp.bitwise_and(d, 127) == lane,
                   1.0, 0.0).astype(jnp.bfloat16)
    s = jnp.where(lax.shift_right_logical(d, 7) == grp,
                  1.0, 0.0).astype(jnp.bfloat16)
    dinv_ref[...] += lax.dot_general(s, m1, (((1,), (1,)), ((), ())),
                                     preferred_element_type=jnp.float32)

    @pl.when(i == ng - 1)
    def _():
        dinv_ref[...] = lax.rsqrt(dinv_ref[...] + 1.0)


@functools.lru_cache(maxsize=None)
def _deg_hist(n_pad, e):
    ng = e // CE
    hrows = n_pad // 128
    return pl.pallas_call(
        functools.partial(_deg_hist_body, ng, hrows, e),
        grid=(ng,),
        in_specs=[pl.BlockSpec((1, 1, CE), lambda i: (i, 0, 0))],
        out_specs=pl.BlockSpec((hrows, 128), lambda i: (0, 0)),
        out_shape=jax.ShapeDtypeStruct((hrows, 128), jnp.float32),
    )


def _dinv_col(dinv_ref):
    rows = _ROWS // 128
    a = dinv_ref[pl.ds(pl.program_id(0) * rows, rows), :]
    rid = lax.broadcasted_iota(jnp.int32, (_ROWS, rows), 0)
    gid = lax.broadcasted_iota(jnp.int32, (_ROWS, rows), 1)
    oh = jnp.where(lax.shift_right_logical(rid, 7) == gid, 1.0, 0.0)
    b = jnp.dot(oh, a, preferred_element_type=jnp.float32)
    rid2 = lax.broadcasted_iota(jnp.int32, (_ROWS, 128), 0)
    lid = lax.broadcasted_iota(jnp.int32, (_ROWS, 128), 1)
    m = jnp.where(jnp.bitwise_and(rid2, 127) == lid, 1.0, 0.0)
    return jnp.sum(b * m, axis=1, keepdims=True)


def _mm_scale_body(dinv_ref, x_ref, w_ref, y_ref):
    xw = jnp.dot(x_ref[...], w_ref[...], preferred_element_type=jnp.float32)
    y_ref[...] = xw * _dinv_col(dinv_ref)


@functools.lru_cache(maxsize=None)
def _mm_scale(n, d, h, n_pad):
    g = n_pad // _ROWS
    hrows = n_pad // 128
    return pl.pallas_call(
        _mm_scale_body,
        grid=(g,),
        in_specs=[
            pl.BlockSpec((hrows, 128), lambda i: (0, 0)),
            pl.BlockSpec((_ROWS, d), lambda i: (i, 0)),
            pl.BlockSpec((d, h), lambda i: (0, 0)),
        ],
        out_specs=pl.BlockSpec((_ROWS, h), lambda i: (i, 0)),
        out_shape=jax.ShapeDtypeStruct((n_pad, h), jnp.float32),
    )


NBUF = 5
ZCOPY = 40
ZWB = 128


@functools.lru_cache(maxsize=None)
def _edge_scatter(n, h, e):
    cpw = e // (NW * CHUNK)
    epw = cpw * CHUNK
    ngrp = cpw // GRP
    n_pad = _pad_rows(n)
    rps = n_pad // NS
    nz = rps // ZCOPY

    def body(y_hbm, src1_hbm, dst4_hbm, acc_out, sidx, didx, b0, b1, b2, b3,
             b4, shared_acc, g0, g1, g2, g3, g4, s0, s1, s2, s3, s4, dsem):
        c = lax.axis_index("c")
        s = lax.axis_index("s")
        wid = s * NC + c
        bufs = (b0, b1, b2, b3, b4)
        gsem = (g0, g1, g2, g3, g4)
        ssem = (s0, s1, s2, s3, s4)

        idx_load = pltpu.async_copy(src1_hbm.at[pl.ds(wid * epw, epw)], sidx,
                                    g0)
        pltpu.sync_copy(dst4_hbm.at[wid, 0], didx.at[0])
        if ngrp > 1:
            pltpu.async_copy(dst4_hbm.at[wid, 1], didx.at[1], dsem)

        @pl.loop(0, CHUNK)
        def _(i):
            for j in range(h // 16):
                b0[i, pl.ds(j * 16, 16)] = jnp.zeros((16,), jnp.float32)

        for k in range(nz):
            pltpu.async_copy(b0.at[pl.ds(0, ZCOPY)],
                             shared_acc.at[pl.ds(s * rps + k * ZCOPY, ZCOPY)],
                             s0)
        for k in range(nz):
            pltpu.make_async_copy(
                b0.at[pl.ds(0, ZCOPY)],
                shared_acc.at[pl.ds(s * rps + k * ZCOPY, ZCOPY)], s0).wait()
        idx_load.wait()

        def gather(jj, p):
            pltpu.async_copy(y_hbm.at[sidx.at[pl.ds(jj * CHUNK, CHUNK)]],
                             bufs[p], gsem[p])

        for p in range(NBUF):
            gather(p, p)
        plsc.subcore_barrier()

        @pl.loop(0, cpw // NBUF)
        def _(i):
            jb = i * NBUF
            for p in range(NBUF):
                j = jb + p
                grp = j // GRP
                r = j - grp * GRP
                par = lax.rem(grp, 2)
                if p == 0:
                    @pl.when((r == 0) & (grp >= 1))
                    def _():
                        pltpu.make_async_copy(dst4_hbm.at[wid, grp],
                                              didx.at[par], dsem).wait()
                if p == 4:
                    @pl.when((r == 4) & (grp + 1 < ngrp))
                    def _():
                        pltpu.async_copy(dst4_hbm.at[wid, grp + 1],
                                        didx.at[1 - par], dsem)
                q = (p + NBUF - 1) % NBUF
                pltpu.make_async_copy(
                    y_hbm.at[sidx.at[pl.ds(j * CHUNK, CHUNK)]],
                    bufs[p], gsem[p]).wait()
                pltpu.async_copy(bufs[p], shared_acc.at[didx.at[par, r]],
                                 ssem[p], add=True)

                @pl.when((j >= 1) & (j + NBUF - 1 < cpw))
                def _():
                    jm = j - 1
                    gm = jm // GRP
                    pltpu.make_async_copy(
                        bufs[q],
                        shared_acc.at[didx.at[lax.rem(gm, 2), jm - gm * GRP]],
                        ssem[q]).wait()
                    gather(j + NBUF - 1, q)

        for p in range(NBUF):
            j = cpw - NBUF + p
            pltpu.make_async_copy(
                bufs[p],
                shared_acc.at[didx.at[(j // GRP) % 2, j % GRP]],
                ssem[p]).wait()

        plsc.subcore_barrier()
        for k in range(rps // ZWB):
            sl = pl.ds(s * rps + k * ZWB, ZWB)
            pltpu.async_copy(shared_acc.at[sl], acc_out.at[c, sl], s0)
        for k in range(rps // ZWB):
            sl = pl.ds(s * rps + k * ZWB, ZWB)
            pltpu.make_async_copy(shared_acc.at[sl], acc_out.at[c, sl],
                                  s0).wait()

    return pl.kernel(
        body,
        out_type=jax.ShapeDtypeStruct((NC, n_pad, h), jnp.float32),
        mesh=_mesh(),
        scratch_types=[
            pltpu.VMEM((epw,), jnp.int32),
            pltpu.VMEM((2, GRP, CHUNK), jnp.int32),
            pltpu.VMEM((CHUNK, h), jnp.float32),
            pltpu.VMEM((CHUNK, h), jnp.float32),
            pltpu.VMEM((CHUNK, h), jnp.float32),
            pltpu.VMEM((CHUNK, h), jnp.float32),
            pltpu.VMEM((CHUNK, h), jnp.float32),
            pltpu.VMEM_SHARED((n_pad, h), jnp.float32),
            pltpu.SemaphoreType.DMA,
            pltpu.SemaphoreType.DMA,
            pltpu.SemaphoreType.DMA,
            pltpu.SemaphoreType.DMA,
            pltpu.SemaphoreType.DMA,
            pltpu.SemaphoreType.DMA,
            pltpu.SemaphoreType.DMA,
            pltpu.SemaphoreType.DMA,
            pltpu.SemaphoreType.DMA,
            pltpu.SemaphoreType.DMA,
            pltpu.SemaphoreType.DMA,
        ],
    )


_ROWS = 2048


def _layer_mid_body(acc_ref, y_ref, dinv_ref, b_ref, w_ref, h_ref, y2_ref):
    dinv = _dinv_col(dinv_ref)
    agg = acc_ref[0] + acc_ref[1] + y_ref[...]
    hpre = agg * dinv + b_ref[...]
    hh = jnp.maximum(hpre, 0.0)
    h_ref[...] = hh
    y2_ref[...] = jnp.dot(hh, w_ref[...], preferred_element_type=jnp.float32) * dinv


@functools.lru_cache(maxsize=None)
def _layer_mid(n, h, n_pad):
    g = n_pad // _ROWS
    hrows = n_pad // 128
    return pl.pallas_call(
        _layer_mid_body,
        grid=(g,),
        in_specs=[
            pl.BlockSpec((NC, _ROWS, h), lambda i: (0, i, 0)),
            pl.BlockSpec((_ROWS, h), lambda i: (i, 0)),
            pl.BlockSpec((hrows, 128), lambda i: (0, 0)),
            pl.BlockSpec((1, h), lambda i: (0, 0)),
            pl.BlockSpec((h, h), lambda i: (0, 0)),
        ],
        out_specs=[
            pl.BlockSpec((_ROWS, h), lambda i: (i, 0)),
            pl.BlockSpec((_ROWS, h), lambda i: (i, 0)),
        ],
        out_shape=[
            jax.ShapeDtypeStruct((n_pad, h), jnp.float32),
            jax.ShapeDtypeStruct((n_pad, h), jnp.float32),
        ],
    )


def _layer_out_body(acc_ref, y2_ref, dinv_ref, b_ref, hres_ref, wl_ref, bl_ref,
                    out_ref):
    dinv = _dinv_col(dinv_ref)
    agg = acc_ref[0] + acc_ref[1] + y2_ref[...]
    h2 = jnp.maximum(agg * dinv + b_ref[...], 0.0) + hres_ref[...]
    logits = jnp.dot(h2, wl_ref[...], preferred_element_type=jnp.float32)
    logits = logits + bl_ref[...]
    m = jnp.max(logits, axis=1, keepdims=True)
    z = logits - m
    lse = jnp.log(jnp.sum(jnp.exp(z), axis=1, keepdims=True))
    out_ref[...] = z - lse


@functools.lru_cache(maxsize=None)
def _layer_out(n, h, cdim, n_pad):
    g = n_pad // _ROWS
    hrows = n_pad // 128
    return pl.pallas_call(
        _layer_out_body,
        grid=(g,),
        in_specs=[
            pl.BlockSpec((NC, _ROWS, h), lambda i: (0, i, 0)),
            pl.BlockSpec((_ROWS, h), lambda i: (i, 0)),
            pl.BlockSpec((hrows, 128), lambda i: (0, 0)),
            pl.BlockSpec((1, h), lambda i: (0, 0)),
            pl.BlockSpec((_ROWS, h), lambda i: (i, 0)),
            pl.BlockSpec((h, cdim), lambda i: (0, 0)),
            pl.BlockSpec((1, cdim), lambda i: (0, 0)),
        ],
        out_specs=pl.BlockSpec((_ROWS, cdim), lambda i: (i, 0)),
        out_shape=jax.ShapeDtypeStruct((n, cdim), jnp.float32),
    )


def kernel(x, edge_index, W1, b1, W2, b2, Wl, bl):
    n, d = x.shape
    h = W1.shape[1]
    cdim = Wl.shape[1]
    e = edge_index.shape[1]
    cpw = e // (NW * CHUNK)
    n_pad = _pad_rows(n)

    src1 = edge_index[0]
    dst1 = edge_index[1]
    dst4 = dst1.reshape(NW, cpw // GRP, GRP, CHUNK)
    dsth = dst1.reshape(e // CE, 1, CE)

    dinv2d = _deg_hist(n_pad, e)(dsth)
    y1 = _mm_scale(n, d, h, n_pad)(dinv2d, x, W1)
    acc1 = _edge_scatter(n, h, e)(y1, src1, dst4)
    h1, y2 = _layer_mid(n, h, n_pad)(acc1, y1, dinv2d, b1.reshape(1, h), W2)
    acc2 = _edge_scatter(n, h, e)(y2, src1, dst4)
    out = _layer_out(n, h, cdim, n_pad)(acc2, y2, dinv2d, b2.reshape(1, h),
                                        h1, Wl, bl.reshape(1, cdim))
    return out
